# Initial kernel scaffold; baseline (speedup 1.0000x reference)
import jax
import jax.numpy as jnp
from jax.experimental import pallas as pl


def kernel(features, edge_index, W1, b1, W2, b2, W3, b3, W4, b4, Wp, bp):
    raise NotImplementedError("write your pallas kernel here")


# speedup vs baseline: 5.2415x; 5.2415x over previous
import functools

import jax
import jax.numpy as jnp
from jax import lax
from jax.experimental import pallas as pl
from jax.experimental.pallas import tpu as pltpu
from jax.experimental.pallas import tpu_sc as plsc

N = 10000
E = 320000
D = 128
NC = 2
NS = 16
NW = NC * NS
EPW = E // NW
K = 80
CH = EPW // K
SROW = 624
ZC = 16
DW = 128
RB = 1000
_F32 = jnp.float32
_PREC = lax.Precision.DEFAULT


def _zero_rows(sh_ref, zbuf_v, row0, last):
  def _zchunk(t, carry):
    pltpu.sync_copy(zbuf_v, sh_ref.at[pl.ds(row0 + t * ZC, ZC)])
    return carry

  lax.fori_loop(0, SROW // ZC, _zchunk, 0)

  @pl.when(last)
  def _():
    pltpu.sync_copy(zbuf_v, sh_ref.at[pl.ds(NS * SROW, N - NS * SROW)])


def _write_rows(sh_ref, out_hbm, cid, row0, last):
  pltpu.sync_copy(sh_ref.at[pl.ds(row0, SROW)], out_hbm.at[cid, pl.ds(row0, SROW)])

  @pl.when(last)
  def _():
    tail0 = NS * SROW
    tail = N - NS * SROW
    pltpu.sync_copy(sh_ref.at[pl.ds(tail0, tail)], out_hbm.at[cid, pl.ds(tail0, tail)])


def _deg_body(src_hbm, dst_hbm, dout_hbm, din_hbm,
              idx_v, ones_v, zbuf_v, acc_sh):
  cid = lax.axis_index("c")
  sid = lax.axis_index("s")
  wid = sid * NC + cid
  row0 = pl.multiple_of(sid * SROW, 8)
  last = sid == NS - 1

  def _init(i, carry):
    for c in range(DW // 16):
      ones_v[i, pl.ds(c * 16, 16)] = jnp.ones((16,), _F32)
      zbuf_v[i, pl.ds(c * 16, 16)] = jnp.zeros((16,), _F32)
    return carry

  lax.fori_loop(0, ZC, _init, 0)

  def _ones(i, carry):
    for c in range(DW // 16):
      ones_v[ZC + i, pl.ds(c * 16, 16)] = jnp.ones((16,), _F32)
    return carry

  lax.fori_loop(0, K - ZC, _ones, 0)

  for idx_hbm, out_hbm in ((src_hbm, dout_hbm), (dst_hbm, din_hbm)):
    _zero_rows(acc_sh, zbuf_v, row0, last)
    plsc.subcore_barrier()

    def _step(j, carry):
      base = pl.multiple_of(wid * EPW + j * K, 8)
      pltpu.sync_copy(idx_hbm.at[pl.ds(base, K)], idx_v)
      pltpu.sync_copy(ones_v, acc_sh.at[idx_v], add=True)
      return carry

    lax.fori_loop(0, CH, _step, 0)
    plsc.subcore_barrier()
    _write_rows(acc_sh, out_hbm, cid, row0, last)
    plsc.subcore_barrier()


def _agg_body(hs_hbm, src_hbm, dst_hbm, out_hbm,
              sidx_v, didx_v, rows_v, zbuf_v, agg_sh, sem):
  cid = lax.axis_index("c")
  sid = lax.axis_index("s")
  wid = sid * NC + cid
  row0 = pl.multiple_of(sid * SROW, 8)
  last = sid == NS - 1

  def _zrow(i, carry):
    for c in range(D // 16):
      zbuf_v[i, pl.ds(c * 16, 16)] = jnp.zeros((16,), _F32)
    return carry

  lax.fori_loop(0, ZC, _zrow, 0)
  _zero_rows(agg_sh, zbuf_v, row0, last)
  plsc.subcore_barrier()

  def _step(j, carry):
    base = pl.multiple_of(wid * EPW + j * K, 8)
    pltpu.sync_copy(src_hbm.at[pl.ds(base, K)], sidx_v)
    cp = pltpu.async_copy(hs_hbm.at[sidx_v], rows_v, sem)
    pltpu.sync_copy(dst_hbm.at[pl.ds(base, K)], didx_v)
    cp.wait()
    pltpu.sync_copy(rows_v, agg_sh.at[didx_v], add=True)
    return carry

  lax.fori_loop(0, CH, _step, 0)
  plsc.subcore_barrier()
  _write_rows(agg_sh, out_hbm, cid, row0, last)


def _sc_mesh():
  return plsc.VectorSubcoreMesh(core_axis_name="c", subcore_axis_name="s",
                                num_cores=NC, num_subcores=NS)


def _sc_degree(src, dst):
  fn = pl.kernel(
      _deg_body,
      out_type=[jax.ShapeDtypeStruct((NC, N, DW), _F32),
                jax.ShapeDtypeStruct((NC, N, DW), _F32)],
      mesh=_sc_mesh(),
      scratch_types=[
          pltpu.VMEM((K,), jnp.int32),
          pltpu.VMEM((K, DW), _F32),
          pltpu.VMEM((ZC, DW), _F32),
          pltpu.VMEM_SHARED((N, DW), _F32),
      ],
  )
  return fn(src, dst)


def _sc_agg(hs, src, dst):
  fn = pl.kernel(
      _agg_body,
      out_type=jax.ShapeDtypeStruct((NC, N, D), _F32),
      mesh=_sc_mesh(),
      scratch_types=[
          pltpu.VMEM((K,), jnp.int32),
          pltpu.VMEM((K,), jnp.int32),
          pltpu.VMEM((K, D), _F32),
          pltpu.VMEM((ZC, D), _F32),
          pltpu.VMEM_SHARED((N, D), _F32),
          pltpu.SemaphoreType.DMA,
      ],
  )
  return fn(hs, src, dst)


def _scale_in(dp_ref):
  d = dp_ref[0, :, 0:1] + dp_ref[1, :, 0:1]
  return lax.rsqrt(jnp.maximum(d, 1.0))


def _tc_first_body(x_ref, w_ref, dop_ref, o_ref):
  h = jnp.dot(x_ref[...], w_ref[...], preferred_element_type=_F32,
              precision=_PREC)
  o_ref[...] = h * _scale_in(dop_ref)


def _tc_mid_body(a_ref, dip_ref, dop_ref, b_ref, w_ref, o_ref):
  a = (a_ref[0] + a_ref[1]) * _scale_in(dip_ref)
  x = jnp.maximum(a + b_ref[...], 0.0)
  h = jnp.dot(x, w_ref[...], preferred_element_type=_F32, precision=_PREC)
  o_ref[...] = h * _scale_in(dop_ref)


def _tc_final_body(a_ref, dip_ref, b_ref, wp_ref, bp_ref, o_ref, acc_ref):
  i = pl.program_id(0)
  a = (a_ref[0] + a_ref[1]) * _scale_in(dip_ref)
  x = jnp.maximum(a + b_ref[...], 0.0)
  part = jnp.sum(x, axis=0, keepdims=True)

  @pl.when(i == 0)
  def _():
    acc_ref[...] = part

  @pl.when(i > 0)
  def _():
    acc_ref[...] = acc_ref[...] + part

  @pl.when(i == pl.num_programs(0) - 1)
  def _():
    hg = acc_ref[...] * (1.0 / N)
    o_ref[...] = jnp.dot(hg, wp_ref[...], preferred_element_type=_F32,
                         precision=_PREC) + bp_ref[...]


def _row_spec(width):
  return pl.BlockSpec((RB, width), lambda i: (i, 0))


def _part_spec(width):
  return pl.BlockSpec((NC, RB, width), lambda i: (0, i, 0))


def _full_spec(shape):
  return pl.BlockSpec(shape, lambda i: tuple(0 for _ in shape))


def _tc_first(x, w, dout_p):
  return pl.pallas_call(
      _tc_first_body,
      grid=(N // RB,),
      in_specs=[_row_spec(D), _full_spec((D, D)), _part_spec(DW)],
      out_specs=_row_spec(D),
      out_shape=jax.ShapeDtypeStruct((N, D), _F32),
  )(x, w, dout_p)


def _tc_mid(agg_p, din_p, dout_p, b, w):
  return pl.pallas_call(
      _tc_mid_body,
      grid=(N // RB,),
      in_specs=[_part_spec(D), _part_spec(DW), _part_spec(DW),
                _full_spec((1, D)), _full_spec((D, D))],
      out_specs=_row_spec(D),
      out_shape=jax.ShapeDtypeStruct((N, D), _F32),
  )(agg_p, din_p, dout_p, b, w)


def _tc_final(agg_p, din_p, b, wp, bp):
  return pl.pallas_call(
      _tc_final_body,
      grid=(N // RB,),
      in_specs=[_part_spec(D), _part_spec(DW), _full_spec((1, D)),
                _full_spec((D, 1)), _full_spec((1, 1))],
      out_specs=_full_spec((1, 1)),
      out_shape=jax.ShapeDtypeStruct((1, 1), _F32),
      scratch_shapes=[pltpu.VMEM((1, D), _F32)],
  )(agg_p, din_p, b, wp, bp)


@jax.jit
def _run(features, src, dst, W1, b1, W2, b2, W3, b3, W4, b4, Wp, bp):
  dout_p, din_p = _sc_degree(src, dst)
  hs = _tc_first(features, W1, dout_p)
  for b, w in ((b1, W2), (b2, W3), (b3, W4)):
    agg_p = _sc_agg(hs, src, dst)
    hs = _tc_mid(agg_p, din_p, dout_p, b.reshape(1, D), w)
  agg_p = _sc_agg(hs, src, dst)
  return _tc_final(agg_p, din_p, b4.reshape(1, D), Wp, bp.reshape(1, 1))


def kernel(features, edge_index, W1, b1, W2, b2, W3, b3, W4, b4, Wp, bp):
  src = edge_index[0]
  dst = edge_index[1]
  return _run(features, src, dst, W1, b1, W2, b2, W3, b3, W4, b4, Wp, bp)


# speedup vs baseline: 6.7419x; 1.2863x over previous
import functools

import jax
import jax.numpy as jnp
from jax import lax
from jax.experimental import pallas as pl
from jax.experimental.pallas import tpu as pltpu
from jax.experimental.pallas import tpu_sc as plsc

N = 10000
E = 320000
D = 128
NC = 2
NS = 16
NW = NC * NS
EPW = E // NW
K = 80
CH = EPW // K
SROW = 624
ZC = 16
DW = 128
RB = 1000
_F32 = jnp.float32
_PREC = lax.Precision.DEFAULT


def _zero_rows(sh_ref, zbuf_v, row0, last):
  def _zchunk(t, carry):
    pltpu.sync_copy(zbuf_v, sh_ref.at[pl.ds(row0 + t * ZC, ZC)])
    return carry

  lax.fori_loop(0, SROW // ZC, _zchunk, 0)

  @pl.when(last)
  def _():
    pltpu.sync_copy(zbuf_v, sh_ref.at[pl.ds(NS * SROW, N - NS * SROW)])


def _write_rows(sh_ref, out_hbm, cid, row0, last):
  pltpu.sync_copy(sh_ref.at[pl.ds(row0, SROW)], out_hbm.at[cid, pl.ds(row0, SROW)])

  @pl.when(last)
  def _():
    tail0 = NS * SROW
    tail = N - NS * SROW
    pltpu.sync_copy(sh_ref.at[pl.ds(tail0, tail)], out_hbm.at[cid, pl.ds(tail0, tail)])


def _deg_body(src_hbm, dst_hbm, dout_hbm, din_hbm,
              idx_v, ones_v, zbuf_v, acc_sh):
  cid = lax.axis_index("c")
  sid = lax.axis_index("s")
  wid = sid * NC + cid
  row0 = pl.multiple_of(sid * SROW, 8)
  last = sid == NS - 1

  def _init(i, carry):
    for c in range(DW // 16):
      ones_v[i, pl.ds(c * 16, 16)] = jnp.ones((16,), _F32)
      zbuf_v[i, pl.ds(c * 16, 16)] = jnp.zeros((16,), _F32)
    return carry

  lax.fori_loop(0, ZC, _init, 0)

  def _ones(i, carry):
    for c in range(DW // 16):
      ones_v[ZC + i, pl.ds(c * 16, 16)] = jnp.ones((16,), _F32)
    return carry

  lax.fori_loop(0, K - ZC, _ones, 0)

  for idx_hbm, out_hbm in ((src_hbm, dout_hbm), (dst_hbm, din_hbm)):
    _zero_rows(acc_sh, zbuf_v, row0, last)
    plsc.subcore_barrier()

    def _step(j, carry):
      base = pl.multiple_of(wid * EPW + j * K, 8)
      pltpu.sync_copy(idx_hbm.at[pl.ds(base, K)], idx_v)
      pltpu.sync_copy(ones_v, acc_sh.at[idx_v], add=True)
      return carry

    lax.fori_loop(0, CH, _step, 0)
    plsc.subcore_barrier()
    _write_rows(acc_sh, out_hbm, cid, row0, last)
    plsc.subcore_barrier()


def _agg_body(hs_hbm, src_hbm, dst_hbm, out_hbm,
              sidx0, sidx1, didx0, didx1, rows0, rows1, zbuf_v, agg_sh,
              sem0, sem1):
  cid = lax.axis_index("c")
  sid = lax.axis_index("s")
  wid = sid * NC + cid
  row0 = pl.multiple_of(sid * SROW, 8)
  last = sid == NS - 1
  e0 = wid * EPW

  def _zrow(i, carry):
    for c in range(D // 16):
      zbuf_v[i, pl.ds(c * 16, 16)] = jnp.zeros((16,), _F32)
    return carry

  lax.fori_loop(0, ZC, _zrow, 0)
  _zero_rows(agg_sh, zbuf_v, row0, last)
  plsc.subcore_barrier()

  def _chunk(j):
    return pl.ds(pl.multiple_of(e0 + j * K, 8), K)

  pltpu.sync_copy(src_hbm.at[_chunk(0)], sidx0)
  pltpu.async_copy(hs_hbm.at[sidx0], rows0, sem0)

  def _pair(jj, carry):
    j = jj * 2
    pltpu.sync_copy(src_hbm.at[_chunk(j + 1)], sidx1)
    pltpu.async_copy(hs_hbm.at[sidx1], rows1, sem1)
    pltpu.sync_copy(dst_hbm.at[_chunk(j)], didx0)
    pltpu.make_async_copy(hs_hbm.at[sidx0], rows0, sem0).wait()
    pltpu.sync_copy(rows0, agg_sh.at[didx0], add=True)

    pltpu.sync_copy(src_hbm.at[_chunk(j + 2)], sidx0)
    pltpu.async_copy(hs_hbm.at[sidx0], rows0, sem0)
    pltpu.sync_copy(dst_hbm.at[_chunk(j + 1)], didx1)
    pltpu.make_async_copy(hs_hbm.at[sidx1], rows1, sem1).wait()
    pltpu.sync_copy(rows1, agg_sh.at[didx1], add=True)
    return carry

  lax.fori_loop(0, (CH - 1) // 2, _pair, 0)

  pltpu.sync_copy(dst_hbm.at[_chunk(CH - 1)], didx0)
  pltpu.make_async_copy(hs_hbm.at[sidx0], rows0, sem0).wait()
  pltpu.sync_copy(rows0, agg_sh.at[didx0], add=True)

  plsc.subcore_barrier()
  _write_rows(agg_sh, out_hbm, cid, row0, last)


def _sc_mesh():
  return plsc.VectorSubcoreMesh(core_axis_name="c", subcore_axis_name="s",
                                num_cores=NC, num_subcores=NS)


def _sc_degree(src, dst):
  fn = pl.kernel(
      _deg_body,
      out_type=[jax.ShapeDtypeStruct((NC, N, DW), _F32),
                jax.ShapeDtypeStruct((NC, N, DW), _F32)],
      mesh=_sc_mesh(),
      scratch_types=[
          pltpu.VMEM((K,), jnp.int32),
          pltpu.VMEM((K, DW), _F32),
          pltpu.VMEM((ZC, DW), _F32),
          pltpu.VMEM_SHARED((N, DW), _F32),
      ],
  )
  return fn(src, dst)


def _sc_agg(hs, src, dst):
  fn = pl.kernel(
      _agg_body,
      out_type=jax.ShapeDtypeStruct((NC, N, D), _F32),
      mesh=_sc_mesh(),
      scratch_types=[
          pltpu.VMEM((K,), jnp.int32),
          pltpu.VMEM((K,), jnp.int32),
          pltpu.VMEM((K,), jnp.int32),
          pltpu.VMEM((K,), jnp.int32),
          pltpu.VMEM((K, D), _F32),
          pltpu.VMEM((K, D), _F32),
          pltpu.VMEM((ZC, D), _F32),
          pltpu.VMEM_SHARED((N, D), _F32),
          pltpu.SemaphoreType.DMA,
          pltpu.SemaphoreType.DMA,
      ],
  )
  return fn(hs, src, dst)


def _scale_in(dp_ref):
  d = dp_ref[0, :, 0:1] + dp_ref[1, :, 0:1]
  return lax.rsqrt(jnp.maximum(d, 1.0))


def _tc_first_body(x_ref, w_ref, dop_ref, o_ref):
  h = jnp.dot(x_ref[...], w_ref[...], preferred_element_type=_F32,
              precision=_PREC)
  o_ref[...] = h * _scale_in(dop_ref)


def _tc_mid_body(a_ref, dip_ref, dop_ref, b_ref, w_ref, o_ref):
  a = (a_ref[0] + a_ref[1]) * _scale_in(dip_ref)
  x = jnp.maximum(a + b_ref[...], 0.0)
  h = jnp.dot(x, w_ref[...], preferred_element_type=_F32, precision=_PREC)
  o_ref[...] = h * _scale_in(dop_ref)


def _tc_final_body(a_ref, dip_ref, b_ref, wp_ref, bp_ref, o_ref, acc_ref):
  i = pl.program_id(0)
  a = (a_ref[0] + a_ref[1]) * _scale_in(dip_ref)
  x = jnp.maximum(a + b_ref[...], 0.0)
  part = jnp.sum(x, axis=0, keepdims=True)

  @pl.when(i == 0)
  def _():
    acc_ref[...] = part

  @pl.when(i > 0)
  def _():
    acc_ref[...] = acc_ref[...] + part

  @pl.when(i == pl.num_programs(0) - 1)
  def _():
    hg = acc_ref[...] * (1.0 / N)
    o_ref[...] = jnp.dot(hg, wp_ref[...], preferred_element_type=_F32,
                         precision=_PREC) + bp_ref[...]


def _row_spec(width):
  return pl.BlockSpec((RB, width), lambda i: (i, 0))


def _part_spec(width):
  return pl.BlockSpec((NC, RB, width), lambda i: (0, i, 0))


def _full_spec(shape):
  return pl.BlockSpec(shape, lambda i: tuple(0 for _ in shape))


def _tc_first(x, w, dout_p):
  return pl.pallas_call(
      _tc_first_body,
      grid=(N // RB,),
      in_specs=[_row_spec(D), _full_spec((D, D)), _part_spec(DW)],
      out_specs=_row_spec(D),
      out_shape=jax.ShapeDtypeStruct((N, D), _F32),
  )(x, w, dout_p)


def _tc_mid(agg_p, din_p, dout_p, b, w):
  return pl.pallas_call(
      _tc_mid_body,
      grid=(N // RB,),
      in_specs=[_part_spec(D), _part_spec(DW), _part_spec(DW),
                _full_spec((1, D)), _full_spec((D, D))],
      out_specs=_row_spec(D),
      out_shape=jax.ShapeDtypeStruct((N, D), _F32),
  )(agg_p, din_p, dout_p, b, w)


def _tc_final(agg_p, din_p, b, wp, bp):
  return pl.pallas_call(
      _tc_final_body,
      grid=(N // RB,),
      in_specs=[_part_spec(D), _part_spec(DW), _full_spec((1, D)),
                _full_spec((D, 1)), _full_spec((1, 1))],
      out_specs=_full_spec((1, 1)),
      out_shape=jax.ShapeDtypeStruct((1, 1), _F32),
      scratch_shapes=[pltpu.VMEM((1, D), _F32)],
  )(agg_p, din_p, b, wp, bp)


@jax.jit
def _run(features, src, dst, W1, b1, W2, b2, W3, b3, W4, b4, Wp, bp):
  dout_p, din_p = _sc_degree(src, dst)
  hs = _tc_first(features, W1, dout_p)
  for b, w in ((b1, W2), (b2, W3), (b3, W4)):
    agg_p = _sc_agg(hs, src, dst)
    hs = _tc_mid(agg_p, din_p, dout_p, b.reshape(1, D), w)
  agg_p = _sc_agg(hs, src, dst)
  return _tc_final(agg_p, din_p, b4.reshape(1, D), Wp, bp.reshape(1, 1))


def kernel(features, edge_index, W1, b1, W2, b2, W3, b3, W4, b4, Wp, bp):
  src = edge_index[0]
  dst = edge_index[1]
  return _run(features, src, dst, W1, b1, W2, b2, W3, b3, W4, b4, Wp, bp)


# speedup vs baseline: 6.8656x; 1.0183x over previous
import functools

import jax
import jax.numpy as jnp
from jax import lax
from jax.experimental import pallas as pl
from jax.experimental.pallas import tpu as pltpu
from jax.experimental.pallas import tpu_sc as plsc

N = 10000
E = 320000
D = 128
NC = 2
NS = 16
NW = NC * NS
EPW = E // NW
K = 80
CH = EPW // K
SROW = 624
ZC = 16
DW = 128
RB = 1000
_F32 = jnp.float32
_PREC = lax.Precision.DEFAULT


def _zero_rows(sh_ref, zbuf_v, row0, last):
  def _zchunk(t, carry):
    pltpu.sync_copy(zbuf_v, sh_ref.at[pl.ds(row0 + t * ZC, ZC)])
    return carry

  lax.fori_loop(0, SROW // ZC, _zchunk, 0)

  @pl.when(last)
  def _():
    pltpu.sync_copy(zbuf_v, sh_ref.at[pl.ds(NS * SROW, N - NS * SROW)])


def _write_rows(sh_ref, out_hbm, cid, row0, last):
  pltpu.sync_copy(sh_ref.at[pl.ds(row0, SROW)], out_hbm.at[cid, pl.ds(row0, SROW)])

  @pl.when(last)
  def _():
    tail0 = NS * SROW
    tail = N - NS * SROW
    pltpu.sync_copy(sh_ref.at[pl.ds(tail0, tail)], out_hbm.at[cid, pl.ds(tail0, tail)])


def _deg_body(src_hbm, dst_hbm, dout_hbm, din_hbm,
              idx_v, ones_v, zbuf_v, acc_sh):
  cid = lax.axis_index("c")
  sid = lax.axis_index("s")
  row0 = pl.multiple_of(sid * SROW, 8)
  last = sid == NS - 1
  eps = E // NS
  ch = eps // K

  def _init(i, carry):
    for c in range(DW // 16):
      ones_v[i, pl.ds(c * 16, 16)] = jnp.ones((16,), _F32)
      zbuf_v[i, pl.ds(c * 16, 16)] = jnp.zeros((16,), _F32)
    return carry

  lax.fori_loop(0, ZC, _init, 0)

  def _ones(i, carry):
    for c in range(DW // 16):
      ones_v[ZC + i, pl.ds(c * 16, 16)] = jnp.ones((16,), _F32)
    return carry

  lax.fori_loop(0, K - ZC, _ones, 0)

  _zero_rows(acc_sh, zbuf_v, row0, last)
  plsc.subcore_barrier()

  for core, idx_hbm in ((0, src_hbm), (1, dst_hbm)):
    @pl.when(cid == core)
    def _():
      def _step(j, carry):
        base = pl.multiple_of(sid * eps + j * K, 8)
        pltpu.sync_copy(idx_hbm.at[pl.ds(base, K)], idx_v)
        pltpu.sync_copy(ones_v, acc_sh.at[idx_v], add=True)
        return carry

      lax.fori_loop(0, ch, _step, 0)

  plsc.subcore_barrier()

  for core, out_hbm in ((0, dout_hbm), (1, din_hbm)):
    @pl.when(cid == core)
    def _():
      pltpu.sync_copy(acc_sh.at[pl.ds(row0, SROW)], out_hbm.at[pl.ds(row0, SROW)])

      @pl.when(last)
      def _():
        tail0 = NS * SROW
        tail = N - NS * SROW
        pltpu.sync_copy(acc_sh.at[pl.ds(tail0, tail)], out_hbm.at[pl.ds(tail0, tail)])


def _agg_body(hs_hbm, src_hbm, dst_hbm, out_hbm,
              sidx0, sidx1, didx0, didx1, rows0, rows1, zbuf_v, agg_sh,
              sem0, sem1):
  cid = lax.axis_index("c")
  sid = lax.axis_index("s")
  wid = sid * NC + cid
  row0 = pl.multiple_of(sid * SROW, 8)
  last = sid == NS - 1
  e0 = wid * EPW

  def _zrow(i, carry):
    for c in range(D // 16):
      zbuf_v[i, pl.ds(c * 16, 16)] = jnp.zeros((16,), _F32)
    return carry

  lax.fori_loop(0, ZC, _zrow, 0)
  _zero_rows(agg_sh, zbuf_v, row0, last)
  plsc.subcore_barrier()

  def _chunk(j):
    return pl.ds(pl.multiple_of(e0 + j * K, 8), K)

  pltpu.sync_copy(src_hbm.at[_chunk(0)], sidx0)
  pltpu.async_copy(hs_hbm.at[sidx0], rows0, sem0)

  def _pair(jj, carry):
    j = jj * 2
    pltpu.sync_copy(src_hbm.at[_chunk(j + 1)], sidx1)
    pltpu.async_copy(hs_hbm.at[sidx1], rows1, sem1)
    pltpu.sync_copy(dst_hbm.at[_chunk(j)], didx0)
    pltpu.make_async_copy(hs_hbm.at[sidx0], rows0, sem0).wait()
    pltpu.sync_copy(rows0, agg_sh.at[didx0], add=True)

    pltpu.sync_copy(src_hbm.at[_chunk(j + 2)], sidx0)
    pltpu.async_copy(hs_hbm.at[sidx0], rows0, sem0)
    pltpu.sync_copy(dst_hbm.at[_chunk(j + 1)], didx1)
    pltpu.make_async_copy(hs_hbm.at[sidx1], rows1, sem1).wait()
    pltpu.sync_copy(rows1, agg_sh.at[didx1], add=True)
    return carry

  lax.fori_loop(0, (CH - 1) // 2, _pair, 0)

  pltpu.sync_copy(dst_hbm.at[_chunk(CH - 1)], didx0)
  pltpu.make_async_copy(hs_hbm.at[sidx0], rows0, sem0).wait()
  pltpu.sync_copy(rows0, agg_sh.at[didx0], add=True)

  plsc.subcore_barrier()
  _write_rows(agg_sh, out_hbm, cid, row0, last)


def _sc_mesh():
  return plsc.VectorSubcoreMesh(core_axis_name="c", subcore_axis_name="s",
                                num_cores=NC, num_subcores=NS)


def _sc_degree(src, dst):
  fn = pl.kernel(
      _deg_body,
      out_type=[jax.ShapeDtypeStruct((N, DW), _F32),
                jax.ShapeDtypeStruct((N, DW), _F32)],
      mesh=_sc_mesh(),
      scratch_types=[
          pltpu.VMEM((K,), jnp.int32),
          pltpu.VMEM((K, DW), _F32),
          pltpu.VMEM((ZC, DW), _F32),
          pltpu.VMEM_SHARED((N, DW), _F32),
      ],
  )
  return fn(src, dst)


def _sc_agg(hs, src, dst):
  fn = pl.kernel(
      _agg_body,
      out_type=jax.ShapeDtypeStruct((NC, N, D), _F32),
      mesh=_sc_mesh(),
      scratch_types=[
          pltpu.VMEM((K,), jnp.int32),
          pltpu.VMEM((K,), jnp.int32),
          pltpu.VMEM((K,), jnp.int32),
          pltpu.VMEM((K,), jnp.int32),
          pltpu.VMEM((K, D), _F32),
          pltpu.VMEM((K, D), _F32),
          pltpu.VMEM((ZC, D), _F32),
          pltpu.VMEM_SHARED((N, D), _F32),
          pltpu.SemaphoreType.DMA,
          pltpu.SemaphoreType.DMA,
      ],
  )
  return fn(hs, src, dst)


def _scale_in(dp_ref):
  return lax.rsqrt(jnp.maximum(dp_ref[:, 0:1], 1.0))


def _tc_first_body(x_ref, w_ref, dop_ref, o_ref):
  h = jnp.dot(x_ref[...], w_ref[...], preferred_element_type=_F32,
              precision=_PREC)
  o_ref[...] = h * _scale_in(dop_ref)


def _tc_mid_body(a_ref, dip_ref, dop_ref, b_ref, w_ref, o_ref):
  a = (a_ref[0] + a_ref[1]) * _scale_in(dip_ref)
  x = jnp.maximum(a + b_ref[...], 0.0)
  h = jnp.dot(x, w_ref[...], preferred_element_type=_F32, precision=_PREC)
  o_ref[...] = h * _scale_in(dop_ref)


def _tc_final_body(a_ref, dip_ref, b_ref, wp_ref, bp_ref, o_ref, acc_ref):
  i = pl.program_id(0)
  a = (a_ref[0] + a_ref[1]) * _scale_in(dip_ref)
  x = jnp.maximum(a + b_ref[...], 0.0)
  part = jnp.sum(x, axis=0, keepdims=True)

  @pl.when(i == 0)
  def _():
    acc_ref[...] = part

  @pl.when(i > 0)
  def _():
    acc_ref[...] = acc_ref[...] + part

  @pl.when(i == pl.num_programs(0) - 1)
  def _():
    hg = acc_ref[...] * (1.0 / N)
    o_ref[...] = jnp.dot(hg, wp_ref[...], preferred_element_type=_F32,
                         precision=_PREC) + bp_ref[...]


def _row_spec(width):
  return pl.BlockSpec((RB, width), lambda i: (i, 0))


def _part_spec(width):
  return pl.BlockSpec((NC, RB, width), lambda i: (0, i, 0))


def _full_spec(shape):
  return pl.BlockSpec(shape, lambda i: tuple(0 for _ in shape))


def _tc_first(x, w, dout_p):
  return pl.pallas_call(
      _tc_first_body,
      grid=(N // RB,),
      in_specs=[_row_spec(D), _full_spec((D, D)), _row_spec(DW)],
      out_specs=_row_spec(D),
      out_shape=jax.ShapeDtypeStruct((N, D), _F32),
  )(x, w, dout_p)


def _tc_mid(agg_p, din_p, dout_p, b, w):
  return pl.pallas_call(
      _tc_mid_body,
      grid=(N // RB,),
      in_specs=[_part_spec(D), _row_spec(DW), _row_spec(DW),
                _full_spec((1, D)), _full_spec((D, D))],
      out_specs=_row_spec(D),
      out_shape=jax.ShapeDtypeStruct((N, D), _F32),
  )(agg_p, din_p, dout_p, b, w)


def _tc_final(agg_p, din_p, b, wp, bp):
  return pl.pallas_call(
      _tc_final_body,
      grid=(N // RB,),
      in_specs=[_part_spec(D), _row_spec(DW), _full_spec((1, D)),
                _full_spec((D, 1)), _full_spec((1, 1))],
      out_specs=_full_spec((1, 1)),
      out_shape=jax.ShapeDtypeStruct((1, 1), _F32),
      scratch_shapes=[pltpu.VMEM((1, D), _F32)],
  )(agg_p, din_p, b, wp, bp)


@jax.jit
def _run(features, src, dst, W1, b1, W2, b2, W3, b3, W4, b4, Wp, bp):
  dout_p, din_p = _sc_degree(src, dst)
  hs = _tc_first(features, W1, dout_p)
  for b, w in ((b1, W2), (b2, W3), (b3, W4)):
    agg_p = _sc_agg(hs, src, dst)
    hs = _tc_mid(agg_p, din_p, dout_p, b.reshape(1, D), w)
  agg_p = _sc_agg(hs, src, dst)
  return _tc_final(agg_p, din_p, b4.reshape(1, D), Wp, bp.reshape(1, 1))


def kernel(features, edge_index, W1, b1, W2, b2, W3, b3, W4, b4, Wp, bp):
  src = edge_index[0]
  dst = edge_index[1]
  return _run(features, src, dst, W1, b1, W2, b2, W3, b3, W4, b4, Wp, bp)


# speedup vs baseline: 7.6231x; 1.1103x over previous
import functools

import jax
import jax.numpy as jnp
from jax import lax
from jax.experimental import pallas as pl
from jax.experimental.pallas import tpu as pltpu
from jax.experimental.pallas import tpu_sc as plsc

N = 10000
E = 320000
D = 128
NC = 2
NS = 16
NW = NC * NS
EPW = E // NW
K = 80
CH = EPW // K
SROW = 624
ZC = 16
DW = 128
RB = 1000
_F32 = jnp.float32
_PREC = lax.Precision.DEFAULT


def _zero_rows(sh_ref, zbuf_v, row0, last):
  def _zchunk(t, carry):
    pltpu.sync_copy(zbuf_v, sh_ref.at[pl.ds(row0 + t * ZC, ZC)])
    return carry

  lax.fori_loop(0, SROW // ZC, _zchunk, 0)

  @pl.when(last)
  def _():
    pltpu.sync_copy(zbuf_v, sh_ref.at[pl.ds(NS * SROW, N - NS * SROW)])


def _write_rows(sh_ref, out_hbm, cid, row0, last):
  pltpu.sync_copy(sh_ref.at[pl.ds(row0, SROW)], out_hbm.at[cid, pl.ds(row0, SROW)])

  @pl.when(last)
  def _():
    tail0 = NS * SROW
    tail = N - NS * SROW
    pltpu.sync_copy(sh_ref.at[pl.ds(tail0, tail)], out_hbm.at[cid, pl.ds(tail0, tail)])


def _deg_body(src_hbm, dst_hbm, dout_hbm, din_hbm,
              idx_v, idx2_v, ones_v, zbuf_v, acc_sh, sem0, sem1):
  cid = lax.axis_index("c")
  sid = lax.axis_index("s")
  row0 = pl.multiple_of(sid * SROW, 8)
  last = sid == NS - 1
  eps = E // NS
  ch = eps // K

  def _init(i, carry):
    for c in range(DW // 16):
      ones_v[i, pl.ds(c * 16, 16)] = jnp.ones((16,), _F32)
      zbuf_v[i, pl.ds(c * 16, 16)] = jnp.zeros((16,), _F32)
    return carry

  lax.fori_loop(0, ZC, _init, 0)

  def _ones(i, carry):
    for c in range(DW // 16):
      ones_v[ZC + i, pl.ds(c * 16, 16)] = jnp.ones((16,), _F32)
    return carry

  lax.fori_loop(0, K - ZC, _ones, 0)

  _zero_rows(acc_sh, zbuf_v, row0, last)
  plsc.subcore_barrier()

  for core, idx_hbm in ((0, src_hbm), (1, dst_hbm)):
    @pl.when(cid == core)
    def _():
      def _chunk(j):
        return pl.ds(pl.multiple_of(sid * eps + j * K, 8), K)

      def _pair(jj, carry):
        j = jj * 2

        @pl.when(jj > 0)
        def _():
          pltpu.make_async_copy(ones_v, acc_sh.at[idx_v], sem0).wait()

        pltpu.sync_copy(idx_hbm.at[_chunk(j)], idx_v)
        pltpu.async_copy(ones_v, acc_sh.at[idx_v], sem0, add=True)

        @pl.when(jj > 0)
        def _():
          pltpu.make_async_copy(ones_v, acc_sh.at[idx2_v], sem1).wait()

        pltpu.sync_copy(idx_hbm.at[_chunk(j + 1)], idx2_v)
        pltpu.async_copy(ones_v, acc_sh.at[idx2_v], sem1, add=True)
        return carry

      lax.fori_loop(0, ch // 2, _pair, 0)
      pltpu.make_async_copy(ones_v, acc_sh.at[idx_v], sem0).wait()
      pltpu.make_async_copy(ones_v, acc_sh.at[idx2_v], sem1).wait()

  plsc.subcore_barrier()

  for core, out_hbm in ((0, dout_hbm), (1, din_hbm)):
    @pl.when(cid == core)
    def _():
      pltpu.sync_copy(acc_sh.at[pl.ds(row0, SROW)], out_hbm.at[pl.ds(row0, SROW)])

      @pl.when(last)
      def _():
        tail0 = NS * SROW
        tail = N - NS * SROW
        pltpu.sync_copy(acc_sh.at[pl.ds(tail0, tail)], out_hbm.at[pl.ds(tail0, tail)])


def _agg_body(hs_hbm, src_hbm, dst_hbm, out_hbm,
              sidx0, sidx1, didx0, didx1, rows0, rows1, zbuf_v, agg_sh,
              sem0, sem1):
  cid = lax.axis_index("c")
  sid = lax.axis_index("s")
  wid = sid * NC + cid
  row0 = pl.multiple_of(sid * SROW, 8)
  last = sid == NS - 1
  e0 = wid * EPW

  def _zrow(i, carry):
    for c in range(D // 16):
      zbuf_v[i, pl.ds(c * 16, 16)] = jnp.zeros((16,), _F32)
    return carry

  lax.fori_loop(0, ZC, _zrow, 0)
  _zero_rows(agg_sh, zbuf_v, row0, last)
  plsc.subcore_barrier()

  def _chunk(j):
    return pl.ds(pl.multiple_of(e0 + j * K, 8), K)

  pltpu.sync_copy(src_hbm.at[_chunk(0)], sidx0)
  pltpu.async_copy(hs_hbm.at[sidx0], rows0, sem0)

  def _pair(jj, carry):
    j = jj * 2
    pltpu.sync_copy(src_hbm.at[_chunk(j + 1)], sidx1)
    pltpu.async_copy(hs_hbm.at[sidx1], rows1, sem1)
    pltpu.sync_copy(dst_hbm.at[_chunk(j)], didx0)
    pltpu.make_async_copy(hs_hbm.at[sidx0], rows0, sem0).wait()
    pltpu.sync_copy(rows0, agg_sh.at[didx0], add=True)

    pltpu.sync_copy(src_hbm.at[_chunk(j + 2)], sidx0)
    pltpu.async_copy(hs_hbm.at[sidx0], rows0, sem0)
    pltpu.sync_copy(dst_hbm.at[_chunk(j + 1)], didx1)
    pltpu.make_async_copy(hs_hbm.at[sidx1], rows1, sem1).wait()
    pltpu.sync_copy(rows1, agg_sh.at[didx1], add=True)
    return carry

  lax.fori_loop(0, (CH - 1) // 2, _pair, 0)

  pltpu.sync_copy(dst_hbm.at[_chunk(CH - 1)], didx0)
  pltpu.make_async_copy(hs_hbm.at[sidx0], rows0, sem0).wait()
  pltpu.sync_copy(rows0, agg_sh.at[didx0], add=True)

  plsc.subcore_barrier()
  _write_rows(agg_sh, out_hbm, cid, row0, last)


def _sc_mesh():
  return plsc.VectorSubcoreMesh(core_axis_name="c", subcore_axis_name="s",
                                num_cores=NC, num_subcores=NS)


def _sc_degree(src, dst):
  fn = pl.kernel(
      _deg_body,
      out_type=[jax.ShapeDtypeStruct((N, DW), _F32),
                jax.ShapeDtypeStruct((N, DW), _F32)],
      mesh=_sc_mesh(),
      scratch_types=[
          pltpu.VMEM((K,), jnp.int32),
          pltpu.VMEM((K,), jnp.int32),
          pltpu.VMEM((K, DW), _F32),
          pltpu.VMEM((ZC, DW), _F32),
          pltpu.VMEM_SHARED((N, DW), _F32),
          pltpu.SemaphoreType.DMA,
          pltpu.SemaphoreType.DMA,
      ],
  )
  return fn(src, dst)


def _sc_agg(hs, src, dst):
  fn = pl.kernel(
      _agg_body,
      out_type=jax.ShapeDtypeStruct((NC, N, D), _F32),
      mesh=_sc_mesh(),
      scratch_types=[
          pltpu.VMEM((K,), jnp.int32),
          pltpu.VMEM((K,), jnp.int32),
          pltpu.VMEM((K,), jnp.int32),
          pltpu.VMEM((K,), jnp.int32),
          pltpu.VMEM((K, D), _F32),
          pltpu.VMEM((K, D), _F32),
          pltpu.VMEM((ZC, D), _F32),
          pltpu.VMEM_SHARED((N, D), _F32),
          pltpu.SemaphoreType.DMA,
          pltpu.SemaphoreType.DMA,
      ],
  )
  return fn(hs, src, dst)


def _scale_in(dp_ref):
  return lax.rsqrt(jnp.maximum(dp_ref[:, 0:1], 1.0))


def _tc_first_body(x_ref, w_ref, dop_ref, o_ref):
  h = jnp.dot(x_ref[...], w_ref[...], preferred_element_type=_F32,
              precision=_PREC)
  o_ref[...] = h * _scale_in(dop_ref)


def _tc_mid_body(a_ref, dip_ref, dop_ref, b_ref, w_ref, o_ref):
  a = (a_ref[0] + a_ref[1]) * _scale_in(dip_ref)
  x = jnp.maximum(a + b_ref[...], 0.0)
  h = jnp.dot(x, w_ref[...], preferred_element_type=_F32, precision=_PREC)
  o_ref[...] = h * _scale_in(dop_ref)


def _tc_final_body(a_ref, dip_ref, b_ref, wp_ref, bp_ref, o_ref, acc_ref):
  i = pl.program_id(0)
  a = (a_ref[0] + a_ref[1]) * _scale_in(dip_ref)
  x = jnp.maximum(a + b_ref[...], 0.0)
  part = jnp.dot(jnp.ones((1, RB), _F32), x, preferred_element_type=_F32,
                 precision=_PREC)

  @pl.when(i == 0)
  def _():
    acc_ref[...] = part

  @pl.when(i > 0)
  def _():
    acc_ref[...] = acc_ref[...] + part

  @pl.when(i == pl.num_programs(0) - 1)
  def _():
    hg = acc_ref[...] * (1.0 / N)
    o_ref[...] = jnp.sum(hg * wp_ref[...], axis=1, keepdims=True) + bp_ref[...]


def _row_spec(width):
  return pl.BlockSpec((RB, width), lambda i: (i, 0))


def _part_spec(width):
  return pl.BlockSpec((NC, RB, width), lambda i: (0, i, 0))


def _full_spec(shape):
  return pl.BlockSpec(shape, lambda i: tuple(0 for _ in shape))


def _tc_first(x, w, dout_p):
  return pl.pallas_call(
      _tc_first_body,
      grid=(N // RB,),
      in_specs=[_row_spec(D), _full_spec((D, D)), _row_spec(DW)],
      out_specs=_row_spec(D),
      out_shape=jax.ShapeDtypeStruct((N, D), _F32),
  )(x, w, dout_p)


def _tc_mid(agg_p, din_p, dout_p, b, w):
  return pl.pallas_call(
      _tc_mid_body,
      grid=(N // RB,),
      in_specs=[_part_spec(D), _row_spec(DW), _row_spec(DW),
                _full_spec((1, D)), _full_spec((D, D))],
      out_specs=_row_spec(D),
      out_shape=jax.ShapeDtypeStruct((N, D), _F32),
  )(agg_p, din_p, dout_p, b, w)


def _tc_final(agg_p, din_p, b, wp, bp):
  return pl.pallas_call(
      _tc_final_body,
      grid=(N // RB,),
      in_specs=[_part_spec(D), _row_spec(DW), _full_spec((1, D)),
                _full_spec((1, D)), _full_spec((1, 1))],
      out_specs=_full_spec((1, 1)),
      out_shape=jax.ShapeDtypeStruct((1, 1), _F32),
      scratch_shapes=[pltpu.VMEM((1, D), _F32)],
  )(agg_p, din_p, b, wp.reshape(1, D), bp)


@jax.jit
def _run(features, src, dst, W1, b1, W2, b2, W3, b3, W4, b4, Wp, bp):
  dout_p, din_p = _sc_degree(src, dst)
  hs = _tc_first(features, W1, dout_p)
  for b, w in ((b1, W2), (b2, W3), (b3, W4)):
    agg_p = _sc_agg(hs, src, dst)
    hs = _tc_mid(agg_p, din_p, dout_p, b.reshape(1, D), w)
  agg_p = _sc_agg(hs, src, dst)
  return _tc_final(agg_p, din_p, b4.reshape(1, D), Wp, bp.reshape(1, 1))


def kernel(features, edge_index, W1, b1, W2, b2, W3, b3, W4, b4, Wp, bp):
  src = edge_index[0]
  dst = edge_index[1]
  return _run(features, src, dst, W1, b1, W2, b2, W3, b3, W4, b4, Wp, bp)
